# baseline (speedup 1.0000x reference)
import functools

import jax
import jax.numpy as jnp
from jax import lax
from jax.experimental import pallas as pl
from jax.experimental.pallas import tpu as pltpu
from jax.experimental.pallas import tpu_sc as plsc

_B = 64
_N = 32768
_K = 256
_L = 16
_NV = _N // _L
_H1 = 2048
_H2 = 2048
_H3 = 1024


def _zero_hist(hist, nbuckets):
    def body(i, carry):
        hist[pl.ds(i * _L, _L)] = jnp.zeros((_L,), jnp.int32)
        return carry
    lax.fori_loop(0, nbuckets // _L, body, 0)


def _scan_hist(hist, nbuckets, remaining):
    lanes = lax.iota(jnp.int32, _L)

    def body(i, carry):
        found, bucket, above, cnt_at, total = carry
        start = nbuckets - _L * (i + 1)
        v = hist[pl.ds(start, _L)]
        rv = lax.rev(v, (0,))
        c = jnp.cumsum(rv)
        cond = (total + c) >= remaining
        ntrue = jnp.sum(cond.astype(jnp.int32))
        this_found = ntrue > 0
        ffs = jnp.int32(_L) - ntrue
        within_above = jnp.sum(jnp.where(cond, 0, rv))
        here = jnp.sum(jnp.where(lanes == ffs, rv, 0))
        use = jnp.logical_and(this_found, found == 0)
        bucket = jnp.where(use, start + (_L - 1) - ffs, bucket)
        above = jnp.where(use, total + within_above, above)
        cnt_at = jnp.where(use, here, cnt_at)
        found = jnp.where(this_found, 1, found)
        total = total + jnp.sum(v)
        return (found, bucket, above, cnt_at, total)

    z = jnp.int32(0)
    _, bucket, above, cnt_at, _ = lax.fori_loop(
        0, nbuckets // _L, body, (z, z, z, z, z))
    return bucket, above, cnt_at


def _process_row(row, y_hbm, out_hbm, yrow, keys, hist):
    pltpu.sync_copy(y_hbm.at[row], yrow)

    ones = jnp.ones((_L,), jnp.int32)
    int_min = jnp.int32(-(2 ** 31))

    _zero_hist(hist, _H1)

    def p1(j, carry):
        y = yrow[pl.ds(j * _L, _L)]
        b = lax.bitcast_convert_type(y, jnp.int32)
        s = b ^ ((b >> 31) & jnp.int32(0x7FFFFFFF))
        s = jnp.where(b == int_min, 0, s)
        keys[pl.ds(j * _L, _L)] = s
        plsc.addupdate_scatter(hist, [(s >> 21) + 1024], ones)
        return carry
    lax.fori_loop(0, _NV, p1, 0)

    b1, above1, _ = _scan_hist(hist, _H1, jnp.int32(_K))
    p1v = b1 - 1024
    rem2 = jnp.int32(_K) - above1

    _zero_hist(hist, _H2)

    def p2(j, carry):
        s = keys[pl.ds(j * _L, _L)]
        msk = (s >> 21) == p1v
        plsc.addupdate_scatter(hist, [(s >> 10) & 2047], ones, mask=msk)
        return carry
    lax.fori_loop(0, _NV, p2, 0)

    b2, above2, _ = _scan_hist(hist, _H2, rem2)
    prefix2 = (p1v << 11) | b2
    rem3 = rem2 - above2

    _zero_hist(hist, _H3)

    def p3(j, carry):
        s = keys[pl.ds(j * _L, _L)]
        msk = (s >> 10) == prefix2
        plsc.addupdate_scatter(hist, [s & 1023], ones, mask=msk)
        return carry
    lax.fori_loop(0, _NV, p3, 0)

    b3, above3, cnt_eq = _scan_hist(hist, _H3, rem3)
    tstar = (prefix2 << 10) | b3
    needed = rem3 - above3

    one_f = jnp.float32(1.0)
    zero_f = jnp.float32(0.0)

    def emit_simple():
        def e(j, carry):
            s = keys[pl.ds(j * _L, _L)]
            yrow[pl.ds(j * _L, _L)] = jnp.where(s >= tstar, one_f, zero_f)
            return carry
        lax.fori_loop(0, _NV, e, 0)

    def emit_tie():
        def e(j, run):
            s = keys[pl.ds(j * _L, _L)]
            eq = s == tstar
            eqi = eq.astype(jnp.int32)
            csum = jnp.cumsum(eqi)
            keep = jnp.logical_and(eq, (run + csum) <= needed)
            m = jnp.logical_or(s > tstar, keep)
            yrow[pl.ds(j * _L, _L)] = jnp.where(m, one_f, zero_f)
            return run + jnp.sum(eqi)
        lax.fori_loop(0, _NV, e, jnp.int32(0))

    lax.cond(cnt_eq == needed, emit_simple, emit_tie)

    pltpu.sync_copy(yrow, out_hbm.at[row])


def _make_sc_kernel(interpret=False):
    mesh = plsc.VectorSubcoreMesh(core_axis_name="c", subcore_axis_name="s",
                                  num_cores=2, num_subcores=16)

    @functools.partial(
        pl.kernel,
        out_type=jax.ShapeDtypeStruct((_B, _N), jnp.float32),
        mesh=mesh,
        scratch_types=[
            pltpu.VMEM((_N,), jnp.float32),
            pltpu.VMEM((_N,), jnp.int32),
            pltpu.VMEM((_H1,), jnp.int32),
        ],
        compiler_params=pltpu.CompilerParams(needs_layout_passes=False),
        interpret=interpret,
    )
    def sc_topk_mask(y_hbm, out_hbm, yrow, keys, hist):
        wid = lax.axis_index("s") * 2 + lax.axis_index("c")
        for r in range(2):
            _process_row(wid * 2 + r, y_hbm, out_hbm, yrow, keys, hist)

    return sc_topk_mask


@jax.jit
def kernel(Yhat):
    return _make_sc_kernel()(Yhat)


# speedup vs baseline: 1.2079x; 1.2079x over previous
import functools

import jax
import jax.numpy as jnp
from jax import lax
from jax.experimental import pallas as pl
from jax.experimental.pallas import tpu as pltpu
from jax.experimental.pallas import tpu_sc as plsc

_B = 64
_N = 32768
_K = 256
_L = 16
_NV = _N // _L
_U = 8
_H1 = 2048
_H2 = 2048
_H3 = 1024


def _zero_hist(hist, nbuckets):
    def body(i, carry):
        for r in range(_U):
            hist[pl.ds((i * _U + r) * _L, _L)] = jnp.zeros((_L,), jnp.int32)
        return carry
    lax.fori_loop(0, nbuckets // (_L * _U), body, 0)


def _scan_hist(hist, nbuckets, remaining):
    def body(i, carry):
        found, fstart, ftotal, total = carry
        base = nbuckets - 4 * _L * (i + 1)
        for r in range(4):
            start = base + (3 - r) * _L
            vt = jnp.sum(hist[pl.ds(start, _L)])
            hit = jnp.logical_and(found == 0, (total + vt) >= remaining)
            fstart = jnp.where(hit, start, fstart)
            ftotal = jnp.where(hit, total, ftotal)
            found = jnp.where(hit, 1, found)
            total = total + vt
        return (found, fstart, ftotal, total)

    z = jnp.int32(0)
    _, fstart, ftotal, _ = lax.fori_loop(
        0, nbuckets // (4 * _L), body, (z, z, z, z))

    lanes = lax.iota(jnp.int32, _L)
    v = hist[pl.ds(fstart, _L)]
    rv = lax.rev(v, (0,))
    c = jnp.cumsum(rv)
    cond = (ftotal + c) >= remaining
    ffs = jnp.int32(_L) - jnp.sum(cond.astype(jnp.int32))
    bucket = fstart + (_L - 1) - ffs
    above = ftotal + jnp.sum(jnp.where(cond, 0, rv))
    cnt_at = jnp.sum(jnp.where(lanes == ffs, rv, 0))
    return bucket, above, cnt_at


def _select_row(buf, keys, hist):
    ones = jnp.ones((_L,), jnp.int32)
    int_min = jnp.int32(-(2 ** 31))

    _zero_hist(hist, _H1)

    def p1(j, carry):
        for r in range(_U):
            ds = pl.ds((j * _U + r) * _L, _L)
            b = lax.bitcast_convert_type(buf[ds], jnp.int32)
            s = b ^ ((b >> 31) & jnp.int32(0x7FFFFFFF))
            s = jnp.where(b == int_min, 0, s)
            keys[ds] = s
            plsc.addupdate_scatter(hist, [(s >> 21) + 1024], ones)
        return carry
    lax.fori_loop(0, _NV // _U, p1, 0)

    b1, above1, _ = _scan_hist(hist, _H1, jnp.int32(_K))
    p1v = b1 - 1024
    rem2 = jnp.int32(_K) - above1

    _zero_hist(hist, _H2)

    def p2(j, carry):
        for r in range(_U):
            s = keys[pl.ds((j * _U + r) * _L, _L)]
            plsc.addupdate_scatter(hist, [(s >> 10) & 2047], ones,
                                   mask=(s >> 21) == p1v)
        return carry
    lax.fori_loop(0, _NV // _U, p2, 0)

    b2, above2, _ = _scan_hist(hist, _H2, rem2)
    prefix2 = (p1v << 11) | b2
    rem3 = rem2 - above2

    _zero_hist(hist, _H3)

    def p3(j, carry):
        for r in range(_U):
            s = keys[pl.ds((j * _U + r) * _L, _L)]
            plsc.addupdate_scatter(hist, [s & 1023], ones,
                                   mask=(s >> 10) == prefix2)
        return carry
    lax.fori_loop(0, _NV // _U, p3, 0)

    b3, above3, cnt_eq = _scan_hist(hist, _H3, rem3)
    tstar = (prefix2 << 10) | b3
    needed = rem3 - above3
    return tstar, needed, cnt_eq


def _emit_row(buf, keys, tstar, needed, cnt_eq):
    one_f = jnp.float32(1.0)
    zero_f = jnp.float32(0.0)

    def emit_simple():
        def e(j, carry):
            for r in range(_U):
                ds = pl.ds((j * _U + r) * _L, _L)
                buf[ds] = jnp.where(keys[ds] >= tstar, one_f, zero_f)
            return carry
        lax.fori_loop(0, _NV // _U, e, 0)

    def emit_tie():
        def e(j, run):
            s = keys[pl.ds(j * _L, _L)]
            eq = s == tstar
            eqi = eq.astype(jnp.int32)
            csum = jnp.cumsum(eqi)
            keep = jnp.logical_and(eq, (run + csum) <= needed)
            m = jnp.logical_or(s > tstar, keep)
            buf[pl.ds(j * _L, _L)] = jnp.where(m, one_f, zero_f)
            return run + jnp.sum(eqi)
        lax.fori_loop(0, _NV, e, jnp.int32(0))

    lax.cond(cnt_eq == needed, emit_simple, emit_tie)


def _make_sc_kernel():
    mesh = plsc.VectorSubcoreMesh(core_axis_name="c", subcore_axis_name="s",
                                  num_cores=2, num_subcores=16)

    @functools.partial(
        pl.kernel,
        out_type=jax.ShapeDtypeStruct((_B, _N), jnp.float32),
        mesh=mesh,
        scratch_types=[
            pltpu.VMEM((_N,), jnp.float32),
            pltpu.VMEM((_N,), jnp.float32),
            pltpu.VMEM((_N,), jnp.int32),
            pltpu.VMEM((_H1,), jnp.int32),
            pltpu.SemaphoreType.DMA,
            pltpu.SemaphoreType.DMA,
        ],
        compiler_params=pltpu.CompilerParams(needs_layout_passes=False),
    )
    def sc_topk_mask(y_hbm, out_hbm, buf_a, buf_b, keys, hist, sem_a, sem_b):
        wid = lax.axis_index("s") * 2 + lax.axis_index("c")
        row0 = wid * 2
        row1 = row0 + 1

        pltpu.async_copy(y_hbm.at[row0], buf_a, sem_a).wait()
        in1 = pltpu.async_copy(y_hbm.at[row1], buf_b, sem_b)

        t0, n0, e0 = _select_row(buf_a, keys, hist)
        _emit_row(buf_a, keys, t0, n0, e0)
        out0 = pltpu.async_copy(buf_a, out_hbm.at[row0], sem_a)

        in1.wait()
        t1, n1, e1 = _select_row(buf_b, keys, hist)
        _emit_row(buf_b, keys, t1, n1, e1)
        out0.wait()
        pltpu.async_copy(buf_b, out_hbm.at[row1], sem_b).wait()

    return sc_topk_mask


@jax.jit
def kernel(Yhat):
    return _make_sc_kernel()(Yhat)


# speedup vs baseline: 3.5950x; 2.9762x over previous
import functools

import jax
import jax.numpy as jnp
from jax import lax
from jax.experimental import pallas as pl
from jax.experimental.pallas import tpu as pltpu
from jax.experimental.pallas import tpu_sc as plsc

_B = 64
_N = 32768
_K = 256
_L = 16
_NV = _N // _L
_H1 = 2048
_H2 = 2048
_H3 = 1024


def _zero_hist(hist, nbuckets):
    @plsc.parallel_loop(0, nbuckets, _L, unroll=8)
    def _(i):
        hist[pl.ds(i, _L)] = jnp.zeros((_L,), jnp.int32)


def _scan_hist(hist, nbuckets, remaining):
    def body(i, carry):
        found, fstart, ftotal, total = carry
        base = nbuckets - 4 * _L * (i + 1)
        for r in range(4):
            start = base + (3 - r) * _L
            vt = jnp.sum(hist[pl.ds(start, _L)])
            hit = jnp.logical_and(found == 0, (total + vt) >= remaining)
            fstart = jnp.where(hit, start, fstart)
            ftotal = jnp.where(hit, total, ftotal)
            found = jnp.where(hit, 1, found)
            total = total + vt
        return (found, fstart, ftotal, total)

    z = jnp.int32(0)
    _, fstart, ftotal, _ = lax.fori_loop(
        0, nbuckets // (4 * _L), body, (z, z, z, z))

    lanes = lax.iota(jnp.int32, _L)
    v = hist[pl.ds(fstart, _L)]
    rv = lax.rev(v, (0,))
    c = jnp.cumsum(rv)
    cond = (ftotal + c) >= remaining
    ffs = jnp.int32(_L) - jnp.sum(cond.astype(jnp.int32))
    bucket = fstart + (_L - 1) - ffs
    above = ftotal + jnp.sum(jnp.where(cond, 0, rv))
    cnt_at = jnp.sum(jnp.where(lanes == ffs, rv, 0))
    return bucket, above, cnt_at


def _select_row(buf, keys, hist):
    ones = jnp.ones((_L,), jnp.int32)
    int_min = jnp.int32(-(2 ** 31))

    _zero_hist(hist, _H1)

    @plsc.parallel_loop(0, _N, _L, unroll=8)
    def _(i):
        ds = pl.ds(i, _L)
        b = lax.bitcast_convert_type(buf[ds], jnp.int32)
        s = b ^ ((b >> 31) & jnp.int32(0x7FFFFFFF))
        s = jnp.where(b == int_min, 0, s)
        keys[ds] = s
        plsc.addupdate_scatter(hist, [(s >> 21) + 1024], ones)

    b1, above1, _ = _scan_hist(hist, _H1, jnp.int32(_K))
    p1v = b1 - 1024
    rem2 = jnp.int32(_K) - above1

    _zero_hist(hist, _H2)

    @plsc.parallel_loop(0, _N, _L, unroll=8)
    def _(i):
        s = keys[pl.ds(i, _L)]
        plsc.addupdate_scatter(hist, [(s >> 10) & 2047], ones,
                               mask=(s >> 21) == p1v)

    b2, above2, _ = _scan_hist(hist, _H2, rem2)
    prefix2 = (p1v << 11) | b2
    rem3 = rem2 - above2

    _zero_hist(hist, _H3)

    @plsc.parallel_loop(0, _N, _L, unroll=8)
    def _(i):
        s = keys[pl.ds(i, _L)]
        plsc.addupdate_scatter(hist, [s & 1023], ones,
                               mask=(s >> 10) == prefix2)

    b3, above3, cnt_eq = _scan_hist(hist, _H3, rem3)
    tstar = (prefix2 << 10) | b3
    needed = rem3 - above3
    return tstar, needed, cnt_eq


def _emit_row(buf, keys, tstar, needed, cnt_eq):
    one_f = jnp.float32(1.0)
    zero_f = jnp.float32(0.0)

    def emit_simple():
        @plsc.parallel_loop(0, _N, _L, unroll=8)
        def _(i):
            ds = pl.ds(i, _L)
            buf[ds] = jnp.where(keys[ds] >= tstar, one_f, zero_f)

    def emit_tie():
        def e(j, run):
            s = keys[pl.ds(j * _L, _L)]
            eq = s == tstar
            eqi = eq.astype(jnp.int32)
            csum = jnp.cumsum(eqi)
            keep = jnp.logical_and(eq, (run + csum) <= needed)
            m = jnp.logical_or(s > tstar, keep)
            buf[pl.ds(j * _L, _L)] = jnp.where(m, one_f, zero_f)
            return run + jnp.sum(eqi)
        lax.fori_loop(0, _NV, e, jnp.int32(0))

    lax.cond(cnt_eq == needed, emit_simple, emit_tie)


def _make_sc_kernel():
    mesh = plsc.VectorSubcoreMesh(core_axis_name="c", subcore_axis_name="s",
                                  num_cores=2, num_subcores=16)

    @functools.partial(
        pl.kernel,
        out_type=jax.ShapeDtypeStruct((_B, _N), jnp.float32),
        mesh=mesh,
        scratch_types=[
            pltpu.VMEM((_N,), jnp.float32),
            pltpu.VMEM((_N,), jnp.float32),
            pltpu.VMEM((_N,), jnp.int32),
            pltpu.VMEM((_H1,), jnp.int32),
            pltpu.SemaphoreType.DMA,
            pltpu.SemaphoreType.DMA,
        ],
        compiler_params=pltpu.CompilerParams(needs_layout_passes=False),
    )
    def sc_topk_mask(y_hbm, out_hbm, buf_a, buf_b, keys, hist, sem_a, sem_b):
        wid = lax.axis_index("s") * 2 + lax.axis_index("c")
        row0 = wid * 2
        row1 = row0 + 1

        pltpu.async_copy(y_hbm.at[row0], buf_a, sem_a).wait()
        in1 = pltpu.async_copy(y_hbm.at[row1], buf_b, sem_b)

        t0, n0, e0 = _select_row(buf_a, keys, hist)
        _emit_row(buf_a, keys, t0, n0, e0)
        out0 = pltpu.async_copy(buf_a, out_hbm.at[row0], sem_a)

        in1.wait()
        t1, n1, e1 = _select_row(buf_b, keys, hist)
        _emit_row(buf_b, keys, t1, n1, e1)
        out0.wait()
        pltpu.async_copy(buf_b, out_hbm.at[row1], sem_b).wait()

    return sc_topk_mask


@jax.jit
def kernel(Yhat):
    return _make_sc_kernel()(Yhat)
